# baseline (speedup 1.0000x reference)
import jax
import jax.numpy as jnp
from jax.experimental import pallas as pl
from jax.experimental.pallas import tpu as pltpu

_B, _C, _K, _L = 16, 144, 128, 256
_C_TIME = 128


def _side_info_body(tab_t_ref, out_ref, scratch, sems):
    ci = jax.lax.broadcasted_iota(jnp.int32, (_C_TIME, _L), 0)
    li = jax.lax.broadcasted_iota(jnp.int32, (_C_TIME, _L), 1)
    c_rem = ci - (ci // 2) * 2
    c_even = (ci - c_rem).astype(jnp.float32)
    ln10000 = 9.210340371976184
    div = jnp.exp(c_even * (-ln10000 / 128.0))
    angle = li.astype(jnp.float32) * div
    pe = jnp.where(c_rem == 0, jnp.sin(angle), jnp.cos(angle))
    scratch[0:_C_TIME, :, :] = jnp.broadcast_to(pe[:, None, :], (_C_TIME, _K, _L))
    tab = tab_t_ref[...]
    scratch[_C_TIME:_C, :, :] = jnp.broadcast_to(tab[:, :, None], (_C - _C_TIME, _K, _L))
    for b in range(_B):
        pltpu.make_async_copy(scratch, out_ref.at[b], sems.at[b]).start()
    for b in range(_B):
        pltpu.make_async_copy(scratch, out_ref.at[b], sems.at[b]).wait()


def _side_info(tab_t):
    return pl.pallas_call(
        _side_info_body,
        in_specs=[pl.BlockSpec((_C - _C_TIME, _K), lambda: (0, 0))],
        out_specs=pl.BlockSpec(memory_space=pl.ANY),
        out_shape=jax.ShapeDtypeStruct((_B, _C, _K, _L), jnp.float32),
        scratch_shapes=[
            pltpu.VMEM((_C, _K, _L), jnp.float32),
            pltpu.SemaphoreType.DMA((_B,)),
        ],
    )(tab_t)


def kernel(cond_mask, table):
    del cond_mask
    tab_t = table.T
    return _side_info(tab_t)
